# baseline (speedup 1.0000x reference)
import functools

import jax
import jax.numpy as jnp
from jax import lax
from jax.experimental import pallas as pl
from jax.experimental.pallas import tpu as pltpu
from jax.experimental.pallas import tpu_sc as plsc

_NC = 2
_NS = 16
_NW = _NC * _NS
_L = 16


def _rowsum_body(emb_ref, out_ref):
    x = emb_ref[...]
    ones = jnp.ones((1, x.shape[1]), jnp.float32)
    r = jax.lax.dot_general(
        ones, x, (((1,), (1,)), ((), ())),
        preferred_element_type=jnp.float32,
    )
    out_ref[...] = jax.nn.sigmoid(r).reshape(out_ref.shape)


def _rowsum_sigmoid(emb_table, blk, v_pad):
    v, d = emb_table.shape
    grid = v_pad // blk
    return pl.pallas_call(
        _rowsum_body,
        grid=(grid,),
        in_specs=[pl.BlockSpec((blk, d), lambda i: (i, 0))],
        out_specs=pl.BlockSpec((blk,), lambda i: (i,)),
        out_shape=jax.ShapeDtypeStruct((v_pad,), jnp.float32),
    )(emb_table)


def _make_sc_gather(v, rows, cols, chunk_rows):
    n_chunks = rows // chunk_rows
    mesh = plsc.VectorSubcoreMesh(core_axis_name="c", subcore_axis_name="s")
    band = 128
    celts = chunk_rows * band

    @functools.partial(
        pl.kernel,
        mesh=mesh,
        out_type=jax.ShapeDtypeStruct((rows, cols), jnp.float32),
        scratch_types=[
            pltpu.VMEM_SHARED((v,), jnp.float32),
            pltpu.VMEM((v,), jnp.float32),
            pltpu.VMEM((chunk_rows, band), jnp.int32),
            pltpu.VMEM((chunk_rows, band), jnp.int32),
            pltpu.VMEM((chunk_rows, band), jnp.float32),
            pltpu.VMEM((chunk_rows, band), jnp.float32),
            pltpu.SemaphoreType.DMA,
            pltpu.SemaphoreType.DMA,
            pltpu.SemaphoreType.DMA,
            pltpu.SemaphoreType.DMA,
            pltpu.SemaphoreType.DMA,
        ],
        compiler_params=pltpu.CompilerParams(needs_layout_passes=False),
    )
    def gather_kernel(s_hbm, idx_hbm, out_hbm, s_sh, s_v, idx_v0, idx_v1,
                      out_v0, out_v1, sem_s, sem_i0, sem_i1, sem_o0, sem_o1):
        sid = lax.axis_index("s")
        wid = sid * _NC + lax.axis_index("c")
        col0 = wid * band
        idx_bufs, out_bufs = [idx_v0, idx_v1], [out_v0, out_v1]
        isems, osems = [sem_i0, sem_i1], [sem_o0, sem_o1]
        icp, ocp = [None] * n_chunks, [None] * n_chunks

        def idx_start(c):
            h = pltpu.make_async_copy(
                idx_hbm.at[pl.ds(c * chunk_rows, chunk_rows),
                           pl.ds(col0, band)],
                idx_bufs[c % 2], isems[c % 2])
            h.start()
            icp[c] = h

        idx_start(0)

        @pl.when(sid == 0)
        def _():
            pltpu.sync_copy(s_hbm, s_sh)

        plsc.subcore_barrier()
        s_h = pltpu.make_async_copy(s_sh, s_v, sem_s)
        s_h.start()
        s_h.wait()

        for c in range(n_chunks):
            b = c % 2
            if c + 1 < n_chunks:
                idx_start(c + 1)
            icp[c].wait()
            if c >= 2:
                ocp[c - 2].wait()
            iv, ov = idx_bufs[b], out_bufs[b]

            @plsc.parallel_loop(0, celts, _L, unroll=2)
            def _(i):
                r = i // band
                sl = pl.ds(i % band, _L)
                ov[r, sl] = plsc.load_gather(s_v, [iv[r, sl]])

            h = pltpu.make_async_copy(
                ov,
                out_hbm.at[pl.ds(c * chunk_rows, chunk_rows),
                           pl.ds(col0, band)],
                osems[b])
            h.start()
            ocp[c] = h

        ocp[n_chunks - 2].wait()
        ocp[n_chunks - 1].wait()

    return gather_kernel


def kernel(idseq, length_list, emb_table):
    b, sl = idseq.shape
    v, _ = emb_table.shape
    v_pad = 102400
    s = _rowsum_sigmoid(emb_table, blk=20480, v_pad=v_pad)
    gather = _make_sc_gather(v_pad, sl, b, chunk_rows=40)
    out_t = gather(s, idseq.T.astype(jnp.int32))
    return out_t.T


# speedup vs baseline: 1.0654x; 1.0654x over previous
import functools

import jax
import jax.numpy as jnp
from jax import lax
from jax.experimental import pallas as pl
from jax.experimental.pallas import tpu as pltpu
from jax.experimental.pallas import tpu_sc as plsc

_NC = 2
_NS = 16
_NW = _NC * _NS
_L = 16


def _rowsum_body(lo_ref, hi_ref, out_ref):
    ones = jnp.ones((1, lo_ref.shape[1]), jnp.float32)

    def sig_row(ref):
        r = jax.lax.dot_general(
            ones, ref[...], (((1,), (1,)), ((), ())),
            preferred_element_type=jnp.float32,
        )
        return jax.nn.sigmoid(r)

    lo = lax.convert_element_type(
        lax.bitcast_convert_type(
            lax.convert_element_type(sig_row(lo_ref), jnp.bfloat16),
            jnp.uint16),
        jnp.uint32)
    hi = lax.convert_element_type(
        lax.bitcast_convert_type(
            lax.convert_element_type(sig_row(hi_ref), jnp.bfloat16),
            jnp.uint16),
        jnp.uint32)
    packed = lax.bitcast_convert_type(lo | (hi << 16), jnp.float32)
    out_ref[...] = packed.reshape(out_ref.shape)


def _rowsum_sigmoid(emb_table, blk, v_pad):
    v, d = emb_table.shape
    half = v_pad // 2
    grid = half // blk
    return pl.pallas_call(
        _rowsum_body,
        grid=(grid,),
        in_specs=[
            pl.BlockSpec((blk, d), lambda i: (i, 0)),
            pl.BlockSpec((blk, d), lambda i, g=grid: (i + g, 0)),
        ],
        out_specs=pl.BlockSpec((blk,), lambda i: (i,)),
        out_shape=jax.ShapeDtypeStruct((half,), jnp.float32),
    )(emb_table, emb_table)


def _make_sc_gather(v_packed, rows, cols, chunk_rows):
    n_chunks = rows // chunk_rows
    mesh = plsc.VectorSubcoreMesh(core_axis_name="c", subcore_axis_name="s")
    band = 128
    celts = chunk_rows * band

    @functools.partial(
        pl.kernel,
        mesh=mesh,
        out_type=jax.ShapeDtypeStruct((rows, cols), jnp.float32),
        scratch_types=[
            pltpu.VMEM_SHARED((v_packed,), jnp.float32),
            pltpu.VMEM((v_packed,), jnp.float32),
            pltpu.VMEM((chunk_rows, band), jnp.int32),
            pltpu.VMEM((chunk_rows, band), jnp.int32),
            pltpu.VMEM((chunk_rows, band), jnp.float32),
            pltpu.VMEM((chunk_rows, band), jnp.float32),
            pltpu.SemaphoreType.DMA,
            pltpu.SemaphoreType.DMA,
            pltpu.SemaphoreType.DMA,
            pltpu.SemaphoreType.DMA,
            pltpu.SemaphoreType.DMA,
        ],
        compiler_params=pltpu.CompilerParams(needs_layout_passes=False),
    )
    def gather_kernel(s_hbm, idx_hbm, out_hbm, s_sh, s_v, idx_v0, idx_v1,
                      out_v0, out_v1, sem_s, sem_i0, sem_i1, sem_o0, sem_o1):
        sid = lax.axis_index("s")
        wid = sid * _NC + lax.axis_index("c")
        col0 = wid * band
        idx_bufs, out_bufs = [idx_v0, idx_v1], [out_v0, out_v1]
        isems, osems = [sem_i0, sem_i1], [sem_o0, sem_o1]
        icp, ocp = [None] * n_chunks, [None] * n_chunks

        def idx_start(c):
            h = pltpu.make_async_copy(
                idx_hbm.at[pl.ds(c * chunk_rows, chunk_rows),
                           pl.ds(col0, band)],
                idx_bufs[c % 2], isems[c % 2])
            h.start()
            icp[c] = h

        idx_start(0)

        @pl.when(sid == 0)
        def _():
            pltpu.sync_copy(s_hbm, s_sh)

        plsc.subcore_barrier()
        s_h = pltpu.make_async_copy(s_sh, s_v, sem_s)
        s_h.start()
        s_h.wait()

        for c in range(n_chunks):
            b = c % 2
            if c + 1 < n_chunks:
                idx_start(c + 1)
            icp[c].wait()
            if c >= 2:
                ocp[c - 2].wait()
            iv, ov = idx_bufs[b], out_bufs[b]

            @plsc.parallel_loop(0, celts, _L, unroll=8)
            def _(i):
                r = i // band
                sl = pl.ds(i % band, _L)
                idx = iv[r, sl]
                m = idx >= v_packed
                w = plsc.load_gather(s_v, [jnp.where(m, idx - v_packed, idx)])
                bits = plsc.bitcast(w, jnp.uint32)
                half16 = jnp.where(m, bits >> 16, bits & jnp.uint32(0xFFFF))
                ov[r, sl] = plsc.bitcast(half16 << 16, jnp.float32)

            h = pltpu.make_async_copy(
                ov,
                out_hbm.at[pl.ds(c * chunk_rows, chunk_rows),
                           pl.ds(col0, band)],
                osems[b])
            h.start()
            ocp[c] = h

        ocp[n_chunks - 2].wait()
        ocp[n_chunks - 1].wait()

    return gather_kernel


def kernel(idseq, length_list, emb_table):
    b, sl = idseq.shape
    v, _ = emb_table.shape
    v_pad = 102400
    s = _rowsum_sigmoid(emb_table, blk=10240, v_pad=v_pad)
    gather = _make_sc_gather(v_pad // 2, sl, b, chunk_rows=40)
    out_t = gather(s, idseq.T.astype(jnp.int32))
    return out_t.T
